# baseline (speedup 1.0000x reference)
import functools

import jax
import jax.numpy as jnp
from jax import lax
from jax.experimental import pallas as pl
from jax.experimental.pallas import tpu as pltpu
from jax.experimental.pallas import tpu_sc as plsc

_NC = 2
_NS = 16
_NW = _NC * _NS
_IG = -1


def _gather3_body(b_per_w, dims,
                  x0, i0, x1, i1, x2, i2,
                  o0, o1, o2,
                  r0, r1, r2,
                  ib0, ib1, ib2, ib3, ib4, ib5, ib6, ib7,
                  big,
                  sem):
    ib = (ib0, ib1, ib2, ib3, ib4, ib5, ib6, ib7)
    wid = lax.axis_index("s") * _NC + lax.axis_index("c")
    base = wid * b_per_w
    pltpu.sync_copy(i0.at[pl.ds(base, b_per_w)], r0)
    pltpu.sync_copy(i1.at[pl.ds(base, b_per_w)], r1)
    pltpu.sync_copy(i2.at[pl.ds(base, b_per_w)], r2)

    nj = b_per_w // 16
    for x, r_v, o, d in ((x0, r0, o0, dims[0]), (x1, r1, o1, dims[1]),
                         (x2, r2, o2, dims[2])):
        p = 128 // d
        shift = p.bit_length() - 1

        def build(jc, _, r_v=r_v, p=p, shift=shift):
            s = pl.ds(jc * 16, 16)
            r = r_v[s]
            rq = lax.shift_right_logical(r, shift)
            rm = lax.rem(r, p)
            for h in range(p):
                ib[h][s] = jnp.where(rm == h, rq, _IG)
            return _

        lax.fori_loop(0, nj, build, 0)

        cps = [pltpu.async_copy(
                   x.at[plsc.Indices(ib[h], ignored_value=_IG)], big, sem)
               for h in range(p)]
        for cp in cps:
            cp.wait()

        def extract(jc, _, r_v=r_v, p=p, d=d):
            s = pl.ds(jc * 16, 16)
            jv = lax.iota(jnp.int32, 16) + jc * 16
            lane0 = lax.rem(r_v[s], p) * d
            for c in range(d):
                val = plsc.load_gather(big, [jv, lane0 + c])
                plsc.store_scatter(big, [jv, jnp.full((16,), c, jnp.int32)],
                                   val)
            return _

        lax.fori_loop(0, nj, extract, 0)
        pltpu.sync_copy(big, o.at[pl.ds(base, b_per_w), :])


def _make_gather3(B, d0, d1, d2):
    b_per_w = B // _NW
    p_max = 128 // min(d0, d1, d2)
    mesh = plsc.VectorSubcoreMesh(core_axis_name="c", subcore_axis_name="s")
    return pl.kernel(
        functools.partial(_gather3_body, b_per_w, (d0, d1, d2)),
        out_type=(
            jax.ShapeDtypeStruct((B, 128), jnp.float32),
            jax.ShapeDtypeStruct((B, 128), jnp.float32),
            jax.ShapeDtypeStruct((B, 128), jnp.float32),
        ),
        mesh=mesh,
        scratch_types=[
            pltpu.VMEM((b_per_w,), jnp.int32),
            pltpu.VMEM((b_per_w,), jnp.int32),
            pltpu.VMEM((b_per_w,), jnp.int32),
            *[pltpu.VMEM((b_per_w,), jnp.int32) for _ in range(p_max)],
            pltpu.VMEM((b_per_w, 128), jnp.float32),
            pltpu.SemaphoreType.DMA,
        ],
        compiler_params=pltpu.CompilerParams(needs_layout_passes=False),
    )


def _mlp_body(x_ref, w_ref, b_ref, o_ref):
    x = jnp.maximum(x_ref[...], 0.0)
    acc = jax.lax.dot_general(
        x, w_ref[...], (((1,), (0,)), ((), ())),
        preferred_element_type=jnp.float32)
    o_ref[...] = jnp.maximum(acc + b_ref[...], 0.0)


def _mlp(x, w, b):
    B, K = x.shape
    N = w.shape[1]
    BLK = 2048
    return pl.pallas_call(
        _mlp_body,
        grid=(B // BLK,),
        in_specs=[
            pl.BlockSpec((BLK, K), lambda i: (i, 0)),
            pl.BlockSpec((K, N), lambda i: (0, 0)),
            pl.BlockSpec((N,), lambda i: (0,)),
        ],
        out_specs=pl.BlockSpec((BLK, N), lambda i: (i, 0)),
        out_shape=jax.ShapeDtypeStruct((B, N), jnp.float32),
    )(x, w, b)


def kernel(arg0_1, arg1_1, arg2_1, arg3_1, arg4_1, arg5_1, arg6_1, arg7_1, arg8_1):
    B = arg1_1.shape[0]
    d0, d1, d2 = arg0_1.shape[1], arg2_1.shape[1], arg4_1.shape[1]
    x0 = arg0_1.reshape(-1, 128)
    x1 = arg2_1.reshape(-1, 128)
    x2 = arg4_1.reshape(-1, 128)
    g = _make_gather3(B, d0, d1, d2)
    f0, f1, f2 = g(x0, arg1_1, x1, arg3_1, x2, arg5_1)
    relu_1 = _mlp(arg7_1, arg6_1, arg8_1)
    return (f0[:, :d0], f1[:, :d1], f2[:, :d2], arg6_1, relu_1)


# speedup vs baseline: 1.4592x; 1.4592x over previous
import functools

import jax
import jax.numpy as jnp
from jax import lax
from jax.experimental import pallas as pl
from jax.experimental.pallas import tpu as pltpu
from jax.experimental.pallas import tpu_sc as plsc

_NC = 2
_NS = 16
_NW = _NC * _NS
_CW = 128
_BATCH = 128


def _gather3_body(B, tabs,
                  t0, i0, t1, i1, t2, i2, tl0, tl1, tl2,
                  o0, o1, o2,
                  idxb, rsel, jsel, jloc, cbuf, ext, jbig, tailb,
                  sem_i, sem_c, sem_s, sem_o):
    wid = lax.axis_index("s") * _NC + lax.axis_index("c")
    padrow = B + wid
    iota = lax.iota(jnp.int32, 16)

    for (t, i, o, tl, d, V, npw, nfull, tw, towner) in (
            (t0, i0, o0, tl0) + tabs[0],
            (t1, i1, o1, tl1) + tabs[1],
            (t2, i2, o2, tl2) + tabs[2]):
        pltpu.async_copy(i, idxb, sem_i).wait()
        lo_w = wid * (npw * _CW)
        hi_w = jnp.minimum(V, lo_w + npw * _CW)

        def select(q, pos, lo_w=lo_w, hi_w=hi_w):
            r = idxb[pl.ds(q * 16, 16)]
            m = jnp.logical_and(r >= lo_w, r < hi_w)
            plsc.store_compressed(rsel.at[pl.ds(pos, 16)], r, mask=m)
            plsc.store_compressed(jsel.at[pl.ds(pos, 16)], iota + q * 16, mask=m)
            return pos + jnp.max(plsc.all_reduce_population_count(m))

        pos = lax.fori_loop(0, B // 16, select, jnp.int32(0))

        def fire(ci, pm, t=t, d=d):
            start = pl.multiple_of(ci * _CW, _CW)
            return pltpu.async_copy(
                t.at[:, pl.ds(start, _CW)],
                cbuf.at[pm, pl.ds(0, d), :], sem_c)

        def wait_chunk(t=t, d=d):
            pltpu.make_async_copy(
                t.at[:, pl.ds(0, _CW)],
                cbuf.at[0, pl.ds(0, d), :], sem_c).wait()

        def flush(bf, pm_e, d=d, o=o):
            return pltpu.async_copy(
                ext.at[pm_e], o.at[plsc.Indices(jbig.at[pm_e])], sem_s)

        def process(ci_local, carry, d=d, o=o, pos=pos, padrow=padrow,
                    lo_w=lo_w):
            bf, pend = carry
            pm = lax.rem(ci_local, 2)
            lo_c = lo_w + ci_local * _CW

            def rescan(q, n, lo_c=lo_c):
                r = rsel[pl.ds(q * 16, 16)]
                j = jsel[pl.ds(q * 16, 16)]
                m = jnp.logical_and(r >= lo_c, r < lo_c + _CW)
                m = jnp.logical_and(m, (q * 16 + iota) < pos)
                plsc.store_compressed(idxb.at[pl.ds(n, 16)], r - lo_c, mask=m)
                plsc.store_compressed(jloc.at[pl.ds(n, 16)], j, mask=m)
                return n + jnp.max(plsc.all_reduce_population_count(m))

            n = lax.fori_loop(0, lax.div(pos + 15, 16), rescan, jnp.int32(0))

            def group(g, carry2, pm=pm, d=d):
                bf2, pend2 = carry2
                pm_b = lax.div(bf2, _BATCH)
                bfo = lax.rem(bf2, _BATCH)

                @pl.when(jnp.logical_and(pend2 == 1, bfo == 0))
                def _():
                    pltpu.make_async_copy(
                        ext.at[0], o.at[plsc.Indices(jbig.at[0])],
                        sem_s).wait()

                pend3 = jnp.where(jnp.logical_and(pend2 == 1, bfo == 0),
                                  0, pend2)
                valid = (g * 16 + iota) < n
                rv = jnp.where(valid, idxb[pl.ds(g * 16, 16)], 0)
                jv = jnp.where(valid, jloc[pl.ds(g * 16, 16)], padrow)
                rows = bfo + iota
                for c in range(d):
                    val = plsc.load_gather(
                        cbuf, [jnp.full((16,), pm, jnp.int32),
                               jnp.full((16,), c, jnp.int32), rv])
                    plsc.store_scatter(
                        ext, [jnp.full((16,), pm_b, jnp.int32), rows,
                              jnp.full((16,), c, jnp.int32)], val)
                plsc.store_scatter(jbig, [jnp.full((16,), pm_b, jnp.int32),
                                          rows], jv)
                bf3 = bf2 + 16

                @pl.when(lax.rem(bf3, _BATCH) == 0)
                def _(pm_b=pm_b):
                    flush(bf3, pm_b)

                pend4 = jnp.where(lax.rem(bf3, _BATCH) == 0, 1, pend3)
                bf4 = lax.rem(bf3, 2 * _BATCH)
                return (bf4, pend4)

            return lax.fori_loop(0, lax.div(n + 15, 16), group, (bf, pend))

        nch_w0 = jnp.minimum(
            jnp.maximum(nfull - wid * npw, 0), npw)

        @pl.when(nch_w0 > 0)
        def _():
            fire(wid * npw, 0)

        def chunk_loop(i, carry, npw=npw):
            @pl.when(i + 1 < nch_w0)
            def _(i=i):
                fire(wid * npw + i + 1, lax.rem(i + 1, 2))

            def do(carry, i=i):
                wait_chunk()
                return process(i, carry)

            return lax.cond(i < nch_w0, do, lambda c: c, carry)

        carry = lax.fori_loop(0, npw, chunk_loop, (jnp.int32(0),
                                                   jnp.int32(0)))
        bf, pend = carry

        if tw > 0:
            @pl.when(wid == towner)
            def _(tl=tl, o=o, d=d, tw=tw, bf=bf, pend=pend, pos=pos,
                  nfull=nfull, flush=flush):
                pltpu.async_copy(tl, tailb.at[pl.ds(0, tw), :],
                                 sem_c).wait()
                lo_c = nfull * _CW

                def rescan(q, n, lo_c=lo_c, tw=tw):
                    r = rsel[pl.ds(q * 16, 16)]
                    j = jsel[pl.ds(q * 16, 16)]
                    m = jnp.logical_and(r >= lo_c, r < lo_c + tw)
                    m = jnp.logical_and(m, (q * 16 + iota) < pos)
                    plsc.store_compressed(idxb.at[pl.ds(n, 16)], r - lo_c,
                                          mask=m)
                    plsc.store_compressed(jloc.at[pl.ds(n, 16)], j, mask=m)
                    return n + jnp.max(plsc.all_reduce_population_count(m))

                n = lax.fori_loop(0, lax.div(pos + 15, 16), rescan,
                                  jnp.int32(0))

                def group(g, carry2, d=d, o=o):
                    bf2, pend2 = carry2
                    pm_b = lax.div(bf2, _BATCH)
                    bfo = lax.rem(bf2, _BATCH)

                    @pl.when(jnp.logical_and(pend2 == 1, bfo == 0))
                    def _():
                        pltpu.make_async_copy(
                            ext.at[0], o.at[plsc.Indices(jbig.at[0])],
                            sem_s).wait()

                    pend3 = jnp.where(jnp.logical_and(pend2 == 1, bfo == 0),
                                      0, pend2)
                    valid = (g * 16 + iota) < n
                    rv = jnp.where(valid, idxb[pl.ds(g * 16, 16)], 0)
                    jv = jnp.where(valid, jloc[pl.ds(g * 16, 16)], padrow)
                    rows = bfo + iota
                    for c in range(d):
                        val = plsc.load_gather(
                            tailb, [rv, jnp.full((16,), c, jnp.int32)])
                        plsc.store_scatter(
                            ext, [jnp.full((16,), pm_b, jnp.int32), rows,
                                  jnp.full((16,), c, jnp.int32)], val)
                    plsc.store_scatter(jbig,
                                       [jnp.full((16,), pm_b, jnp.int32),
                                        rows], jv)
                    bf3 = bf2 + 16

                    @pl.when(lax.rem(bf3, _BATCH) == 0)
                    def _(pm_b=pm_b):
                        flush(bf3, pm_b)

                    pend4 = jnp.where(lax.rem(bf3, _BATCH) == 0, 1, pend3)
                    return (lax.rem(bf3, 2 * _BATCH), pend4)

                bf_t, pend_t = lax.fori_loop(0, lax.div(n + 15, 16), group,
                                             (bf, pend))
                _final_flush(o, ext, jbig, sem_s, padrow, bf_t, pend_t,
                             iota)

            @pl.when(wid != towner)
            def _(o=o, bf=bf, pend=pend):
                _final_flush(o, ext, jbig, sem_s, padrow, bf, pend, iota)
        else:
            _final_flush(o, ext, jbig, sem_s, padrow, bf, pend, iota)


def _final_flush(o, ext, jbig, sem_s, padrow, bf, pend, iota):
    bfo = lax.rem(bf, _BATCH)
    pm_b = lax.div(bf, _BATCH)

    @pl.when(bfo > 0)
    def _():
        def pad(g, _):
            rows = g * 16 + iota
            m = rows >= bfo
            plsc.store_scatter(jbig,
                               [jnp.full((16,), pm_b, jnp.int32), rows],
                               jnp.full((16,), padrow, jnp.int32), mask=m)
            return _

        lax.fori_loop(0, _BATCH // 16, pad, 0)

        @pl.when(pend == 1)
        def _():
            pltpu.make_async_copy(
                ext.at[0], o.at[plsc.Indices(jbig.at[0])], sem_s).wait()

        pltpu.async_copy(ext.at[pm_b], o.at[plsc.Indices(jbig.at[pm_b])],
                         sem_s).wait()

    @pl.when(jnp.logical_and(bfo == 0, pend == 1))
    def _():
        pltpu.make_async_copy(
            ext.at[0], o.at[plsc.Indices(jbig.at[0])], sem_s).wait()


def _chunk_plan(V):
    nfull = V // _CW
    tw = V - nfull * _CW
    npw = -(-nfull // _NW)
    towner = nfull // npw if tw > 0 else 0
    return npw, nfull, tw, towner


def _make_gather3(B, d0, V0, d1, V1, d2, V2):
    tabs = tuple((d, V) + _chunk_plan(V)
                 for d, V in ((d0, V0), (d1, V1), (d2, V2)))
    mesh = plsc.VectorSubcoreMesh(core_axis_name="c", subcore_axis_name="s")
    max_tw = max(t[4] for t in tabs) if any(t[4] for t in tabs) else 8
    return pl.kernel(
        functools.partial(_gather3_body, B, tabs),
        out_type=(
            jax.ShapeDtypeStruct((B + _NW, 128), jnp.float32),
            jax.ShapeDtypeStruct((B + _NW, 128), jnp.float32),
            jax.ShapeDtypeStruct((B + _NW, 128), jnp.float32),
        ),
        mesh=mesh,
        scratch_types=[
            pltpu.VMEM((B,), jnp.int32),
            pltpu.VMEM((B,), jnp.int32),
            pltpu.VMEM((B,), jnp.int32),
            pltpu.VMEM((B,), jnp.int32),
            pltpu.VMEM((2, 64, _CW), jnp.float32),
            pltpu.VMEM((2, _BATCH, 128), jnp.float32),
            pltpu.VMEM((2, _BATCH), jnp.int32),
            pltpu.VMEM((max_tw, 128), jnp.float32),
            pltpu.SemaphoreType.DMA,
            pltpu.SemaphoreType.DMA,
            pltpu.SemaphoreType.DMA,
            pltpu.SemaphoreType.DMA,
        ],
        compiler_params=pltpu.CompilerParams(needs_layout_passes=False),
    )


def _mlp_body(x_ref, w_ref, b_ref, o_ref):
    x = jnp.maximum(x_ref[...], 0.0)
    acc = jax.lax.dot_general(
        x, w_ref[...], (((1,), (0,)), ((), ())),
        preferred_element_type=jnp.float32)
    o_ref[...] = jnp.maximum(acc + b_ref[...], 0.0)


def _mlp(x, w, b):
    B, K = x.shape
    N = w.shape[1]
    BLK = 2048
    return pl.pallas_call(
        _mlp_body,
        grid=(B // BLK,),
        in_specs=[
            pl.BlockSpec((BLK, K), lambda i: (i, 0)),
            pl.BlockSpec((K, N), lambda i: (0, 0)),
            pl.BlockSpec((N,), lambda i: (0,)),
        ],
        out_specs=pl.BlockSpec((BLK, N), lambda i: (i, 0)),
        out_shape=jax.ShapeDtypeStruct((B, N), jnp.float32),
    )(x, w, b)


def _tail_pad(x):
    nfull = x.shape[0] // _CW
    d = x.shape[1]
    tail = x[nfull * _CW:, :]
    return jnp.pad(tail, ((0, 0), (0, 128 - d)))


def kernel(arg0_1, arg1_1, arg2_1, arg3_1, arg4_1, arg5_1, arg6_1, arg7_1, arg8_1):
    B = arg1_1.shape[0]
    d0, d1, d2 = arg0_1.shape[1], arg2_1.shape[1], arg4_1.shape[1]
    g = _make_gather3(B, d0, arg0_1.shape[0], d1, arg2_1.shape[0],
                      d2, arg4_1.shape[0])
    f0, f1, f2 = g(arg0_1.T, arg1_1, arg2_1.T, arg3_1, arg4_1.T, arg5_1,
                   _tail_pad(arg0_1), _tail_pad(arg2_1), _tail_pad(arg4_1))
    relu_1 = _mlp(arg7_1, arg6_1, arg8_1)
    return (f0[:B, :d0], f1[:B, :d1], f2[:B, :d2], arg6_1, relu_1)


# speedup vs baseline: 3.0868x; 2.1154x over previous
import functools

import jax
import jax.numpy as jnp
from jax import lax
from jax.experimental import pallas as pl
from jax.experimental.pallas import tpu as pltpu
from jax.experimental.pallas import tpu_sc as plsc

_NC = 2
_NS = 16
_NW = _NC * _NS
_SUP = 8
_BATCH = 64


def _gather3_body(B, tabs,
                  t0, i0, t1, i1, t2, i2, tl0, tl1, tl2,
                  o0, o1, o2,
                  idxb, rsel, jsel, jloc, cbuf, ext, jbig, tailb,
                  sem_i, sem_c, sem_s, sem_o):
    wid = lax.axis_index("s") * _NC + lax.axis_index("c")
    padrow = B + wid
    iota = lax.iota(jnp.int32, 16)

    def drain(o):
        pltpu.make_async_copy(
            ext.at[0], o.at[plsc.Indices(jbig.at[0])], sem_s).wait()

    def flush_blk(o, bf, pend):
        pm_b = lax.rem(lax.div(bf, _BATCH), 2)
        bfo = lax.rem(bf, _BATCH)

        def pad(g, _):
            rows = g * 16 + iota
            m = jnp.logical_and(rows >= bfo, rows < _BATCH)
            plsc.store_scatter(jbig, [jnp.full((16,), pm_b, jnp.int32),
                                      rows],
                               jnp.full((16,), padrow, jnp.int32), mask=m)
            return _

        lax.fori_loop(0, _BATCH // 16, pad, 0)
        pltpu.async_copy(ext.at[pm_b], o.at[plsc.Indices(jbig.at[pm_b])],
                         sem_s)

        @pl.when(pend == 1)
        def _():
            drain(o)

    def group_step(o, bf, pend, m, jv, load_vals):
        need = lax.rem(bf, _BATCH) > (_BATCH - 16)

        @pl.when(need)
        def _():
            flush_blk(o, bf, pend)

        bf = jnp.where(need, (lax.div(bf, _BATCH) + 1) * _BATCH, bf)
        pend = jnp.where(need, 1, pend)

        pm_b = lax.rem(lax.div(bf, _BATCH), 2)
        bfo = lax.rem(bf, _BATCH)
        mi = m.astype(jnp.int32)
        rank = plsc.cumsum(mi) - mi
        rows = bfo + rank
        pmbv = jnp.full((16,), pm_b, jnp.int32)
        for c, val in load_vals():
            plsc.store_scatter(ext, [pmbv, rows,
                                     jnp.full((16,), c, jnp.int32)],
                               val, mask=m)
        plsc.store_scatter(jbig, [pmbv, rows], jv, mask=m)
        cnt = jnp.max(plsc.all_reduce_population_count(m))
        bf2 = bf + cnt
        full = jnp.logical_and(lax.rem(bf2, _BATCH) == 0, cnt > 0)

        @pl.when(full)
        def _(pm_b=pm_b):
            pltpu.async_copy(ext.at[pm_b],
                             o.at[plsc.Indices(jbig.at[pm_b])], sem_s)

            @pl.when(pend == 1)
            def _():
                drain(o)

        pend = jnp.where(full, 1, pend)
        return bf2, pend

    for (t, i, o, tl, d, V, cw, npw, nfull, tw, towner) in (
            (t0, i0, o0, tl0) + tabs[0],
            (t1, i1, o1, tl1) + tabs[1],
            (t2, i2, o2, tl2) + tabs[2]):
        pltpu.async_copy(i, idxb, sem_i).wait()
        lo_w = wid * (npw * cw)
        hi_w = jnp.minimum(V, lo_w + npw * cw)

        def select(q, pos, lo_w=lo_w, hi_w=hi_w):
            r = idxb[pl.ds(q * 16, 16)]
            m = jnp.logical_and(r >= lo_w, r < hi_w)
            plsc.store_compressed(rsel.at[pl.ds(pos, 16)], r, mask=m)
            plsc.store_compressed(jsel.at[pl.ds(pos, 16)], iota + q * 16,
                                  mask=m)
            return pos + jnp.max(plsc.all_reduce_population_count(m))

        pos = lax.fori_loop(0, B // 16, select, jnp.int32(0), unroll=2)

        nch = jnp.minimum(jnp.maximum(nfull - wid * npw, 0), npw)

        def fire(ci, pm, t=t, d=d, cw=cw):
            start = pl.multiple_of(ci * cw, 128)
            return pltpu.async_copy(
                t.at[:, pl.ds(start, cw)],
                cbuf.at[pm, pl.ds(0, d), pl.ds(0, cw)], sem_c)

        def wait_chunk(t=t, d=d, cw=cw):
            pltpu.make_async_copy(
                t.at[:, pl.ds(0, cw)],
                cbuf.at[0, pl.ds(0, d), pl.ds(0, cw)], sem_c).wait()

        @pl.when(nch > 0)
        def _():
            fire(wid * npw, 0)

        def super_loop(s, carry, lo_w=lo_w, pos=pos, o=o, d=d, cw=cw,
                       npw=npw, nch=nch):
            sup_lo = lo_w + s * (_SUP * cw)
            sup_hi = sup_lo + _SUP * cw

            def rescan(q, n, sup_lo=sup_lo, sup_hi=sup_hi, pos=pos):
                r = rsel[pl.ds(q * 16, 16)]
                j = jsel[pl.ds(q * 16, 16)]
                m = jnp.logical_and(r >= sup_lo, r < sup_hi)
                m = jnp.logical_and(m, (q * 16 + iota) < pos)
                plsc.store_compressed(idxb.at[pl.ds(n, 16)], r, mask=m)
                plsc.store_compressed(jloc.at[pl.ds(n, 16)], j, mask=m)
                return n + jnp.max(plsc.all_reduce_population_count(m))

            sup_n = lax.cond(
                s * _SUP < nch,
                lambda _: lax.fori_loop(0, lax.div(pos + 15, 16), rescan,
                                        jnp.int32(0)),
                lambda _: jnp.int32(0), 0)

            def chunk_loop(k, carry2, s=s, sup_n=sup_n, npw=npw, nch=nch,
                           o=o, cw=cw, d=d):
                il = s * _SUP + k
                ci = wid * npw + il

                @pl.when(il + 1 < nch)
                def _(il=il, ci=ci):
                    fire(ci + 1, lax.rem(il + 1, 2))

                def do(carry3, il=il, ci=ci, o=o, cw=cw, sup_n=sup_n, d=d):
                    wait_chunk()
                    pm = lax.rem(il, 2)
                    lo_c = ci * cw

                    def group(g, c4, pm=pm, lo_c=lo_c, sup_n=sup_n, o=o,
                              cw=cw, d=d):
                        bf, pend = c4
                        r = idxb[pl.ds(g * 16, 16)]
                        j = jloc[pl.ds(g * 16, 16)]
                        m = jnp.logical_and(r >= lo_c, r < lo_c + cw)
                        m = jnp.logical_and(m, (g * 16 + iota) < sup_n)
                        rv = jnp.where(m, r - lo_c, 0)
                        jv = jnp.where(m, j, padrow)

                        def load_vals(rv=rv, pm=pm, d=d):
                            for c in range(d):
                                yield c, plsc.load_gather(
                                    cbuf,
                                    [jnp.full((16,), pm, jnp.int32),
                                     jnp.full((16,), c, jnp.int32), rv])

                        return group_step(o, bf, pend, m, jv, load_vals)

                    return lax.fori_loop(0, lax.div(sup_n + 15, 16),
                                         group, carry3)

                return lax.cond(il < nch, do, lambda c: c, carry2)

            return lax.fori_loop(0, _SUP, chunk_loop, carry)

        nsup = -(-npw // _SUP)
        carry = lax.fori_loop(0, nsup, super_loop,
                              (jnp.int32(0), jnp.int32(0)))
        bf, pend = carry

        if tw > 0:
            @pl.when(wid == towner)
            def _(tl=tl, o=o, d=d, tw=tw, bf=bf, pend=pend, pos=pos,
                  nfull=nfull, cw=cw):
                pltpu.async_copy(tl, tailb.at[pl.ds(0, tw), :],
                                 sem_c).wait()
                lo_c = nfull * cw

                def rescan(q, n, lo_c=lo_c, tw=tw, pos=pos):
                    r = rsel[pl.ds(q * 16, 16)]
                    j = jsel[pl.ds(q * 16, 16)]
                    m = jnp.logical_and(r >= lo_c, r < lo_c + tw)
                    m = jnp.logical_and(m, (q * 16 + iota) < pos)
                    plsc.store_compressed(idxb.at[pl.ds(n, 16)], r - lo_c,
                                          mask=m)
                    plsc.store_compressed(jloc.at[pl.ds(n, 16)], j, mask=m)
                    return n + jnp.max(plsc.all_reduce_population_count(m))

                n = lax.fori_loop(0, lax.div(pos + 15, 16), rescan,
                                  jnp.int32(0))

                def group(g, c4, n=n, o=o, d=d):
                    bf2, pend2 = c4
                    r = idxb[pl.ds(g * 16, 16)]
                    j = jloc[pl.ds(g * 16, 16)]
                    m = (g * 16 + iota) < n
                    rv = jnp.where(m, r, 0)
                    jv = jnp.where(m, j, padrow)

                    def load_vals(rv=rv, d=d):
                        for c in range(d):
                            yield c, plsc.load_gather(
                                tailb, [rv, jnp.full((16,), c, jnp.int32)])

                    return group_step(o, bf2, pend2, m, jv, load_vals)

                bf_t, pend_t = lax.fori_loop(0, lax.div(n + 15, 16),
                                             group, (bf, pend))
                _final(o, bf_t, pend_t, flush_blk, drain)

            @pl.when(wid != towner)
            def _(o=o, bf=bf, pend=pend):
                _final(o, bf, pend, flush_blk, drain)
        else:
            _final(o, bf, pend, flush_blk, drain)


def _final(o, bf, pend, flush_blk, drain):
    bfo = lax.rem(bf, _BATCH)

    @pl.when(bfo > 0)
    def _():
        flush_blk(o, bf, pend)
        drain(o)

    @pl.when(jnp.logical_and(bfo == 0, pend == 1))
    def _():
        drain(o)


def _chunk_plan(V, cw):
    nfull = V // cw
    tw = V - nfull * cw
    npw = -(-nfull // _NW)
    towner = nfull // npw if tw > 0 else 0
    return cw, npw, nfull, tw, towner


def _make_gather3(B, d0, V0, d1, V1, d2, V2):
    tabs = tuple((d, V) + _chunk_plan(V, cw)
                 for d, V, cw in ((d0, V0, 256), (d1, V1, 256),
                                  (d2, V2, 128)))
    max_tw = max(t[5] for t in tabs)
    max_cw = max(t[2] for t in tabs)
    mesh = plsc.VectorSubcoreMesh(core_axis_name="c", subcore_axis_name="s")
    return pl.kernel(
        functools.partial(_gather3_body, B, tabs),
        out_type=(
            jax.ShapeDtypeStruct((B + _NW, 128), jnp.float32),
            jax.ShapeDtypeStruct((B + _NW, 128), jnp.float32),
            jax.ShapeDtypeStruct((B + _NW, 128), jnp.float32),
        ),
        mesh=mesh,
        scratch_types=[
            pltpu.VMEM((B,), jnp.int32),
            pltpu.VMEM((B,), jnp.int32),
            pltpu.VMEM((B,), jnp.int32),
            pltpu.VMEM((B,), jnp.int32),
            pltpu.VMEM((2, 64, max_cw), jnp.float32),
            pltpu.VMEM((2, _BATCH, 128), jnp.float32),
            pltpu.VMEM((2, _BATCH), jnp.int32),
            pltpu.VMEM((max_tw, 128), jnp.float32),
            pltpu.SemaphoreType.DMA,
            pltpu.SemaphoreType.DMA,
            pltpu.SemaphoreType.DMA,
            pltpu.SemaphoreType.DMA,
        ],
        compiler_params=pltpu.CompilerParams(needs_layout_passes=False),
    )


def _mlp_body(x_ref, w_ref, b_ref, o_ref):
    x = jnp.maximum(x_ref[...], 0.0)
    acc = jax.lax.dot_general(
        x, w_ref[...], (((1,), (0,)), ((), ())),
        preferred_element_type=jnp.float32)
    o_ref[...] = jnp.maximum(acc + b_ref[...], 0.0)


def _mlp(x, w, b):
    B, K = x.shape
    N = w.shape[1]
    BLK = 2048
    return pl.pallas_call(
        _mlp_body,
        grid=(B // BLK,),
        in_specs=[
            pl.BlockSpec((BLK, K), lambda i: (i, 0)),
            pl.BlockSpec((K, N), lambda i: (0, 0)),
            pl.BlockSpec((N,), lambda i: (0,)),
        ],
        out_specs=pl.BlockSpec((BLK, N), lambda i: (i, 0)),
        out_shape=jax.ShapeDtypeStruct((B, N), jnp.float32),
    )(x, w, b)


def _tail_pad(x, cw):
    nfull = x.shape[0] // cw
    d = x.shape[1]
    tail = x[nfull * cw:, :]
    return jnp.pad(tail, ((0, 0), (0, 128 - d)))


def kernel(arg0_1, arg1_1, arg2_1, arg3_1, arg4_1, arg5_1, arg6_1, arg7_1, arg8_1):
    B = arg1_1.shape[0]
    d0, d1, d2 = arg0_1.shape[1], arg2_1.shape[1], arg4_1.shape[1]
    g = _make_gather3(B, d0, arg0_1.shape[0], d1, arg2_1.shape[0],
                      d2, arg4_1.shape[0])
    f0, f1, f2 = g(arg0_1.T, arg1_1, arg2_1.T, arg3_1, arg4_1.T, arg5_1,
                   _tail_pad(arg0_1, 256), _tail_pad(arg2_1, 256),
                   _tail_pad(arg4_1, 128))
    relu_1 = _mlp(arg7_1, arg6_1, arg8_1)
    return (f0[:B, :d0], f1[:B, :d1], f2[:B, :d2], arg6_1, relu_1)


# speedup vs baseline: 3.6491x; 1.1822x over previous
import functools

import jax
import jax.numpy as jnp
from jax import lax
from jax.experimental import pallas as pl
from jax.experimental.pallas import tpu as pltpu
from jax.experimental.pallas import tpu_sc as plsc

_NC = 2
_NS = 16
_NW = _NC * _NS
_SUP = 8
_BATCH = 64


def _gather3_body(B, tabs,
                  t0, i0, t1, i1, t2, i2, tl0, tl1, tl2,
                  o0, o1, o2,
                  idxb, selb, cbuf, ext, jbig,
                  sem_i, sem_c, sem_s, sem_o):
    wid = lax.axis_index("s") * _NC + lax.axis_index("c")
    padrow = B + wid
    iota = lax.iota(jnp.int32, 16)

    def drain(o):
        pltpu.make_async_copy(
            ext.at[0], o.at[plsc.Indices(jbig.at[0])], sem_s).wait()

    def flush_blk(o, bf, pend):
        pm_b = lax.rem(lax.div(bf, _BATCH), 2)
        bfo = lax.rem(bf, _BATCH)

        def pad(g, _):
            rows = g * 16 + iota
            m = jnp.logical_and(rows >= bfo, rows < _BATCH)
            plsc.store_scatter(jbig, [jnp.full((16,), pm_b, jnp.int32),
                                      rows],
                               jnp.full((16,), padrow, jnp.int32), mask=m)
            return _

        lax.fori_loop(0, _BATCH // 16, pad, 0)
        pltpu.async_copy(ext.at[pm_b], o.at[plsc.Indices(jbig.at[pm_b])],
                         sem_s)

        @pl.when(pend == 1)
        def _():
            drain(o)

    def group_step(o, bf, pend, m, jv, load_vals):
        need = lax.rem(bf, _BATCH) > (_BATCH - 16)

        @pl.when(need)
        def _():
            flush_blk(o, bf, pend)

        bf = jnp.where(need, (lax.div(bf, _BATCH) + 1) * _BATCH, bf)
        pend = jnp.where(need, 1, pend)

        pm_b = lax.rem(lax.div(bf, _BATCH), 2)
        bfo = lax.rem(bf, _BATCH)
        mi = m.astype(jnp.int32)
        rank = plsc.cumsum(mi) - mi
        rows = bfo + rank
        pmbv = jnp.full((16,), pm_b, jnp.int32)
        for c, val in load_vals():
            plsc.store_scatter(ext, [pmbv, rows,
                                     jnp.full((16,), c, jnp.int32)],
                               val, mask=m)
        plsc.store_scatter(jbig, [pmbv, rows], jv, mask=m)
        cnt = jnp.max(plsc.all_reduce_population_count(m))
        bf2 = bf + cnt
        full = jnp.logical_and(lax.rem(bf2, _BATCH) == 0, cnt > 0)

        @pl.when(full)
        def _(pm_b=pm_b):
            pltpu.async_copy(ext.at[pm_b],
                             o.at[plsc.Indices(jbig.at[pm_b])], sem_s)

            @pl.when(pend == 1)
            def _():
                drain(o)

        pend = jnp.where(full, 1, pend)
        return bf2, pend

    for (t, i, o, tl, d, V, cw, npw, nfull, tw, towner) in (
            (t0, i0, o0, tl0) + tabs[0],
            (t1, i1, o1, tl1) + tabs[1],
            (t2, i2, o2, tl2) + tabs[2]):
        pltpu.async_copy(i, idxb, sem_i).wait()
        lo_w = wid * (npw * cw)
        hi_w = jnp.minimum(V, lo_w + npw * cw)

        def select(q, pos, lo_w=lo_w, hi_w=hi_w):
            r = idxb[pl.ds(q * 16, 16)]
            m = jnp.logical_and(r >= lo_w, r < hi_w)
            packed = lax.shift_left(r - lo_w, 14) + (iota + q * 16)
            plsc.store_compressed(selb.at[pl.ds(pos, 16)], packed, mask=m)
            return pos + jnp.max(plsc.all_reduce_population_count(m))

        pos = lax.fori_loop(0, B // 16, select, jnp.int32(0), unroll=2)

        nch = jnp.minimum(jnp.maximum(nfull - wid * npw, 0), npw)

        def fire(ci, pm, t=t, d=d, cw=cw):
            start = pl.multiple_of(ci * cw, 128)
            return pltpu.async_copy(
                t.at[:, pl.ds(start, cw)],
                cbuf.at[pm, pl.ds(0, d), pl.ds(0, cw)], sem_c)

        def wait_chunk(t=t, d=d, cw=cw):
            pltpu.make_async_copy(
                t.at[:, pl.ds(0, cw)],
                cbuf.at[0, pl.ds(0, d), pl.ds(0, cw)], sem_c).wait()

        @pl.when(nch > 0)
        def _():
            fire(wid * npw, 0)

        @pl.when(nch > 1)
        def _():
            fire(wid * npw + 1, 1)

        def super_loop(s, carry, lo_w=lo_w, pos=pos, o=o, d=d, cw=cw,
                       npw=npw, nch=nch):
            sup_lo = s * (_SUP * cw)
            sup_hi = sup_lo + _SUP * cw

            def rescan(q, n, sup_lo=sup_lo, sup_hi=sup_hi, pos=pos):
                e = selb[pl.ds(q * 16, 16)]
                rl = lax.shift_right_logical(e, 14)
                m = jnp.logical_and(rl >= sup_lo, rl < sup_hi)
                m = jnp.logical_and(m, (q * 16 + iota) < pos)
                plsc.store_compressed(idxb.at[pl.ds(n, 16)], e, mask=m)
                return n + jnp.max(plsc.all_reduce_population_count(m))

            sup_n = lax.cond(
                s * _SUP < nch,
                lambda _: lax.fori_loop(0, lax.div(pos + 15, 16), rescan,
                                        jnp.int32(0)),
                lambda _: jnp.int32(0), 0)

            def chunk_loop(k, carry2, s=s, sup_n=sup_n, npw=npw, nch=nch,
                           o=o, cw=cw, d=d):
                il = s * _SUP + k
                ci = wid * npw + il

                @pl.when(il + 2 < nch)
                def _(il=il, ci=ci):
                    fire(ci + 2, lax.rem(il + 2, 3))

                def do(carry3, il=il, ci=ci, o=o, cw=cw, sup_n=sup_n, d=d):
                    wait_chunk()
                    pm = lax.rem(il, 3)
                    lo_c = il * cw

                    def group(g, c4, pm=pm, lo_c=lo_c, sup_n=sup_n, o=o,
                              cw=cw, d=d):
                        bf, pend = c4
                        e = idxb[pl.ds(g * 16, 16)]
                        rl = lax.shift_right_logical(e, 14)
                        j = lax.bitwise_and(e, 16383)
                        m = jnp.logical_and(rl >= lo_c, rl < lo_c + cw)
                        m = jnp.logical_and(m, (g * 16 + iota) < sup_n)
                        rv = jnp.where(m, rl - lo_c, 0)
                        jv = jnp.where(m, j, padrow)

                        def load_vals(rv=rv, pm=pm, d=d):
                            for c in range(d):
                                yield c, plsc.load_gather(
                                    cbuf,
                                    [jnp.full((16,), pm, jnp.int32),
                                     jnp.full((16,), c, jnp.int32), rv])

                        return group_step(o, bf, pend, m, jv, load_vals)

                    return lax.fori_loop(0, lax.div(sup_n + 15, 16),
                                         group, carry3)

                return lax.cond(il < nch, do, lambda c: c, carry2)

            return lax.fori_loop(0, _SUP, chunk_loop, carry)

        nsup = -(-npw // _SUP)
        carry = lax.fori_loop(0, nsup, super_loop,
                              (jnp.int32(0), jnp.int32(0)))
        bf, pend = carry

        if tw > 0:
            @pl.when(wid == towner)
            def _(tl=tl, o=o, d=d, tw=tw, bf=bf, pend=pend, pos=pos,
                  nfull=nfull, cw=cw, lo_w=lo_w):
                pltpu.async_copy(tl, cbuf.at[0, pl.ds(0, tw), pl.ds(0, 128)],
                                 sem_c).wait()
                lo_c = nfull * cw - lo_w

                def rescan(q, n, lo_c=lo_c, tw=tw, pos=pos):
                    e = selb[pl.ds(q * 16, 16)]
                    rl = lax.shift_right_logical(e, 14)
                    m = jnp.logical_and(rl >= lo_c, rl < lo_c + tw)
                    m = jnp.logical_and(m, (q * 16 + iota) < pos)
                    plsc.store_compressed(idxb.at[pl.ds(n, 16)], e, mask=m)
                    return n + jnp.max(plsc.all_reduce_population_count(m))

                n = lax.fori_loop(0, lax.div(pos + 15, 16), rescan,
                                  jnp.int32(0))

                def group(g, c4, n=n, o=o, d=d, lo_c=lo_c):
                    bf2, pend2 = c4
                    e = idxb[pl.ds(g * 16, 16)]
                    rl = lax.shift_right_logical(e, 14)
                    j = lax.bitwise_and(e, 16383)
                    m = (g * 16 + iota) < n
                    rv = jnp.where(m, rl - lo_c, 0)
                    jv = jnp.where(m, j, padrow)

                    def load_vals(rv=rv, d=d):
                        for c in range(d):
                            yield c, plsc.load_gather(
                                cbuf, [jnp.zeros((16,), jnp.int32), rv,
                                       jnp.full((16,), c, jnp.int32)])

                    return group_step(o, bf2, pend2, m, jv, load_vals)

                bf_t, pend_t = lax.fori_loop(0, lax.div(n + 15, 16),
                                             group, (bf, pend))
                _final(o, bf_t, pend_t, flush_blk, drain)

            @pl.when(wid != towner)
            def _(o=o, bf=bf, pend=pend):
                _final(o, bf, pend, flush_blk, drain)
        else:
            _final(o, bf, pend, flush_blk, drain)


def _final(o, bf, pend, flush_blk, drain):
    bfo = lax.rem(bf, _BATCH)

    @pl.when(bfo > 0)
    def _():
        flush_blk(o, bf, pend)
        drain(o)

    @pl.when(jnp.logical_and(bfo == 0, pend == 1))
    def _():
        drain(o)


def _chunk_plan(V, cw):
    nfull = V // cw
    tw = V - nfull * cw
    npw = -(-nfull // _NW)
    towner = nfull // npw if tw > 0 else 0
    return cw, npw, nfull, tw, towner


def _make_gather3(B, d0, V0, d1, V1, d2, V2):
    tabs = tuple((d, V) + _chunk_plan(V, cw)
                 for d, V, cw in ((d0, V0, 256), (d1, V1, 256),
                                  (d2, V2, 128)))
    max_cw = max(t[2] for t in tabs)
    mesh = plsc.VectorSubcoreMesh(core_axis_name="c", subcore_axis_name="s")
    return pl.kernel(
        functools.partial(_gather3_body, B, tabs),
        out_type=(
            jax.ShapeDtypeStruct((B + _NW, 128), jnp.float32),
            jax.ShapeDtypeStruct((B + _NW, 128), jnp.float32),
            jax.ShapeDtypeStruct((B + _NW, 128), jnp.float32),
        ),
        mesh=mesh,
        scratch_types=[
            pltpu.VMEM((B,), jnp.int32),
            pltpu.VMEM((B,), jnp.int32),
            pltpu.VMEM((3, 64, max_cw), jnp.float32),
            pltpu.VMEM((2, _BATCH, 128), jnp.float32),
            pltpu.VMEM((2, _BATCH), jnp.int32),
            pltpu.SemaphoreType.DMA,
            pltpu.SemaphoreType.DMA,
            pltpu.SemaphoreType.DMA,
            pltpu.SemaphoreType.DMA,
        ],
        compiler_params=pltpu.CompilerParams(needs_layout_passes=False),
    )


def _mlp_body(x_ref, w_ref, b_ref, o_ref):
    x = jnp.maximum(x_ref[...], 0.0)
    acc = jax.lax.dot_general(
        x, w_ref[...], (((1,), (0,)), ((), ())),
        preferred_element_type=jnp.float32)
    o_ref[...] = jnp.maximum(acc + b_ref[...], 0.0)


def _mlp(x, w, b):
    B, K = x.shape
    N = w.shape[1]
    BLK = 2048
    return pl.pallas_call(
        _mlp_body,
        grid=(B // BLK,),
        in_specs=[
            pl.BlockSpec((BLK, K), lambda i: (i, 0)),
            pl.BlockSpec((K, N), lambda i: (0, 0)),
            pl.BlockSpec((N,), lambda i: (0,)),
        ],
        out_specs=pl.BlockSpec((BLK, N), lambda i: (i, 0)),
        out_shape=jax.ShapeDtypeStruct((B, N), jnp.float32),
    )(x, w, b)


def _tail_pad(x, cw):
    nfull = x.shape[0] // cw
    d = x.shape[1]
    tail = x[nfull * cw:, :]
    return jnp.pad(tail, ((0, 0), (0, 128 - d)))


def kernel(arg0_1, arg1_1, arg2_1, arg3_1, arg4_1, arg5_1, arg6_1, arg7_1, arg8_1):
    B = arg1_1.shape[0]
    d0, d1, d2 = arg0_1.shape[1], arg2_1.shape[1], arg4_1.shape[1]
    g = _make_gather3(B, d0, arg0_1.shape[0], d1, arg2_1.shape[0],
                      d2, arg4_1.shape[0])
    f0, f1, f2 = g(arg0_1.T, arg1_1, arg2_1.T, arg3_1, arg4_1.T, arg5_1,
                   _tail_pad(arg0_1, 256), _tail_pad(arg2_1, 256),
                   _tail_pad(arg4_1, 128))
    relu_1 = _mlp(arg7_1, arg6_1, arg8_1)
    return (f0[:B, :d0], f1[:B, :d1], f2[:B, :d2], arg6_1, relu_1)


# speedup vs baseline: 3.8942x; 1.0672x over previous
import functools

import jax
import jax.numpy as jnp
from jax import lax
from jax.experimental import pallas as pl
from jax.experimental.pallas import tpu as pltpu
from jax.experimental.pallas import tpu_sc as plsc

_NC = 2
_NS = 16
_NW = _NC * _NS
_SUP = 8
_BATCH = 64


def _gather3_body(B, tabs,
                  t0, i0, t1, i1, t2, i2, tl0, tl1, tl2,
                  o0, o1, o2,
                  idxb, selb, cbuf, ext, jbig,
                  sem_i, sem_c, sem_s, sem_o):
    wid = lax.axis_index("s") * _NC + lax.axis_index("c")
    padrow = B + wid
    iota = lax.iota(jnp.int32, 16)

    def drain(o):
        pltpu.make_async_copy(
            ext.at[0], o.at[plsc.Indices(jbig.at[0])], sem_s).wait()

    def flush_blk(o, bf, pend):
        pm_b = lax.rem(lax.div(bf, _BATCH), 2)
        bfo = lax.rem(bf, _BATCH)

        def pad(g, _):
            rows = g * 16 + iota
            m = jnp.logical_and(rows >= bfo, rows < _BATCH)
            plsc.store_scatter(jbig, [jnp.full((16,), pm_b, jnp.int32),
                                      rows],
                               jnp.full((16,), padrow, jnp.int32), mask=m)
            return _

        lax.fori_loop(0, _BATCH // 16, pad, 0)
        pltpu.async_copy(ext.at[pm_b], o.at[plsc.Indices(jbig.at[pm_b])],
                         sem_s)

        @pl.when(pend == 1)
        def _():
            drain(o)

    def group_step(o, bf, pend, m, jv, load_vals):
        need = lax.rem(bf, _BATCH) > (_BATCH - 16)

        @pl.when(need)
        def _():
            flush_blk(o, bf, pend)

        bf = jnp.where(need, (lax.div(bf, _BATCH) + 1) * _BATCH, bf)
        pend = jnp.where(need, 1, pend)

        pm_b = lax.rem(lax.div(bf, _BATCH), 2)
        bfo = lax.rem(bf, _BATCH)
        mi = m.astype(jnp.int32)
        rank = plsc.cumsum(mi) - mi
        rows = bfo + rank
        pmbv = jnp.full((16,), pm_b, jnp.int32)
        for c, val in load_vals():
            plsc.store_scatter(ext, [pmbv, rows,
                                     jnp.full((16,), c, jnp.int32)],
                               val, mask=m)
        plsc.store_scatter(jbig, [pmbv, rows], jv, mask=m)
        cnt = jnp.max(plsc.all_reduce_population_count(m))
        bf2 = bf + cnt
        full = jnp.logical_and(lax.rem(bf2, _BATCH) == 0, cnt > 0)

        @pl.when(full)
        def _(pm_b=pm_b):
            pltpu.async_copy(ext.at[pm_b],
                             o.at[plsc.Indices(jbig.at[pm_b])], sem_s)

            @pl.when(pend == 1)
            def _():
                drain(o)

        pend = jnp.where(full, 1, pend)
        return bf2, pend

    for (t, i, o, tl, d, V, cw, npw, nfull, tw, towner) in (
            (t0, i0, o0, tl0) + tabs[0],
            (t1, i1, o1, tl1) + tabs[1],
            (t2, i2, o2, tl2) + tabs[2]):
        pltpu.async_copy(i, idxb, sem_i).wait()
        lo_w = wid * (npw * cw)
        hi_w = jnp.minimum(V, lo_w + npw * cw)

        def select(q, pos, lo_w=lo_w, hi_w=hi_w):
            r = idxb[pl.ds(q * 16, 16)]
            m = jnp.logical_and(r >= lo_w, r < hi_w)
            packed = lax.shift_left(r - lo_w, 14) + (iota + q * 16)
            plsc.store_compressed(selb.at[pl.ds(pos, 16)], packed, mask=m)
            return pos + jnp.max(plsc.all_reduce_population_count(m))

        pos = lax.fori_loop(0, B // 16, select, jnp.int32(0), unroll=2)

        nch = jnp.minimum(jnp.maximum(nfull - wid * npw, 0), npw)

        def fire(ci, pm, t=t, d=d, cw=cw):
            start = pl.multiple_of(ci * cw, 128)
            return pltpu.async_copy(
                t.at[:, pl.ds(start, cw)],
                cbuf.at[pm, pl.ds(0, d), pl.ds(0, cw)], sem_c)

        def wait_chunk(t=t, d=d, cw=cw):
            pltpu.make_async_copy(
                t.at[:, pl.ds(0, cw)],
                cbuf.at[0, pl.ds(0, d), pl.ds(0, cw)], sem_c).wait()

        @pl.when(nch > 0)
        def _():
            fire(wid * npw, 0)

        @pl.when(nch > 1)
        def _():
            fire(wid * npw + 1, 1)

        def super_loop(s, carry, lo_w=lo_w, pos=pos, o=o, d=d, cw=cw,
                       npw=npw, nch=nch):
            sup_lo = s * (_SUP * cw)
            sup_hi = sup_lo + _SUP * cw

            def rescan(q, n, sup_lo=sup_lo, sup_hi=sup_hi, pos=pos):
                e = selb[pl.ds(q * 16, 16)]
                rl = lax.shift_right_logical(e, 14)
                m = jnp.logical_and(rl >= sup_lo, rl < sup_hi)
                m = jnp.logical_and(m, (q * 16 + iota) < pos)
                plsc.store_compressed(idxb.at[pl.ds(n, 16)], e, mask=m)
                return n + jnp.max(plsc.all_reduce_population_count(m))

            sup_n = lax.cond(
                s * _SUP < nch,
                lambda _: lax.fori_loop(0, lax.div(pos + 15, 16), rescan,
                                        jnp.int32(0)),
                lambda _: jnp.int32(0), 0)

            def chunk_loop(k, carry2, s=s, sup_n=sup_n, npw=npw, nch=nch,
                           o=o, cw=cw, d=d):
                il = s * _SUP + k
                ci = wid * npw + il

                @pl.when(il + 2 < nch)
                def _(il=il, ci=ci):
                    fire(ci + 2, lax.rem(il + 2, 3))

                def do(carry3, il=il, ci=ci, o=o, cw=cw, sup_n=sup_n, d=d):
                    wait_chunk()
                    pm = lax.rem(il, 3)
                    lo_c = il * cw

                    def group(g, c4, pm=pm, lo_c=lo_c, sup_n=sup_n, o=o,
                              cw=cw, d=d):
                        bf, pend = c4
                        e = idxb[pl.ds(g * 16, 16)]
                        rl = lax.shift_right_logical(e, 14)
                        j = lax.bitwise_and(e, 16383)
                        m = jnp.logical_and(rl >= lo_c, rl < lo_c + cw)
                        m = jnp.logical_and(m, (g * 16 + iota) < sup_n)
                        rv = jnp.where(m, rl - lo_c, 0)
                        jv = jnp.where(m, j, padrow)

                        def load_vals(rv=rv, pm=pm, d=d):
                            for c in range(d):
                                yield c, plsc.load_gather(
                                    cbuf,
                                    [jnp.full((16,), pm, jnp.int32),
                                     jnp.full((16,), c, jnp.int32), rv])

                        return group_step(o, bf, pend, m, jv, load_vals)

                    return lax.fori_loop(0, lax.div(sup_n + 15, 16),
                                         group, carry3)

                return lax.cond(il < nch, do, lambda c: c, carry2)

            return lax.fori_loop(0, _SUP, chunk_loop, carry)

        nsup = -(-npw // _SUP)
        carry = lax.fori_loop(0, nsup, super_loop,
                              (jnp.int32(0), jnp.int32(0)))
        bf, pend = carry

        if tw > 0:
            @pl.when(wid == towner)
            def _(tl=tl, o=o, d=d, tw=tw, bf=bf, pend=pend, pos=pos,
                  nfull=nfull, cw=cw, lo_w=lo_w):
                pltpu.async_copy(tl, cbuf.at[0, pl.ds(0, tw), pl.ds(0, 128)],
                                 sem_c).wait()
                lo_c = nfull * cw - lo_w

                def rescan(q, n, lo_c=lo_c, tw=tw, pos=pos):
                    e = selb[pl.ds(q * 16, 16)]
                    rl = lax.shift_right_logical(e, 14)
                    m = jnp.logical_and(rl >= lo_c, rl < lo_c + tw)
                    m = jnp.logical_and(m, (q * 16 + iota) < pos)
                    plsc.store_compressed(idxb.at[pl.ds(n, 16)], e, mask=m)
                    return n + jnp.max(plsc.all_reduce_population_count(m))

                n = lax.fori_loop(0, lax.div(pos + 15, 16), rescan,
                                  jnp.int32(0))

                def group(g, c4, n=n, o=o, d=d, lo_c=lo_c):
                    bf2, pend2 = c4
                    e = idxb[pl.ds(g * 16, 16)]
                    rl = lax.shift_right_logical(e, 14)
                    j = lax.bitwise_and(e, 16383)
                    m = (g * 16 + iota) < n
                    rv = jnp.where(m, rl - lo_c, 0)
                    jv = jnp.where(m, j, padrow)

                    def load_vals(rv=rv, d=d):
                        for c in range(d):
                            yield c, plsc.load_gather(
                                cbuf, [jnp.zeros((16,), jnp.int32), rv,
                                       jnp.full((16,), c, jnp.int32)])

                    return group_step(o, bf2, pend2, m, jv, load_vals)

                bf_t, pend_t = lax.fori_loop(0, lax.div(n + 15, 16),
                                             group, (bf, pend))
                _final(o, bf_t, pend_t, flush_blk, drain)

            @pl.when(wid != towner)
            def _(o=o, bf=bf, pend=pend):
                _final(o, bf, pend, flush_blk, drain)
        else:
            _final(o, bf, pend, flush_blk, drain)


def _final(o, bf, pend, flush_blk, drain):
    bfo = lax.rem(bf, _BATCH)

    @pl.when(bfo > 0)
    def _():
        flush_blk(o, bf, pend)
        drain(o)

    @pl.when(jnp.logical_and(bfo == 0, pend == 1))
    def _():
        drain(o)


def _chunk_plan(V, cw):
    nfull = V // cw
    tw = V - nfull * cw
    npw = -(-nfull // _NW)
    towner = nfull // npw if tw > 0 else 0
    return cw, npw, nfull, tw, towner


def _make_gather3(B, d0, V0, d1, V1, d2, V2):
    tabs = tuple((d, V) + _chunk_plan(V, cw)
                 for d, V, cw in ((d0, V0, 384), (d1, V1, 384),
                                  (d2, V2, 128)))
    max_cw = max(t[2] for t in tabs)
    mesh = plsc.VectorSubcoreMesh(core_axis_name="c", subcore_axis_name="s")
    return pl.kernel(
        functools.partial(_gather3_body, B, tabs),
        out_type=(
            jax.ShapeDtypeStruct((B + _NW, 128), jnp.float32),
            jax.ShapeDtypeStruct((B + _NW, 128), jnp.float32),
            jax.ShapeDtypeStruct((B + _NW, 128), jnp.float32),
        ),
        mesh=mesh,
        scratch_types=[
            pltpu.VMEM((B,), jnp.int32),
            pltpu.VMEM((B,), jnp.int32),
            pltpu.VMEM((3, 64, max_cw), jnp.float32),
            pltpu.VMEM((2, _BATCH, 128), jnp.float32),
            pltpu.VMEM((2, _BATCH), jnp.int32),
            pltpu.SemaphoreType.DMA,
            pltpu.SemaphoreType.DMA,
            pltpu.SemaphoreType.DMA,
            pltpu.SemaphoreType.DMA,
        ],
        compiler_params=pltpu.CompilerParams(needs_layout_passes=False),
    )


def _mlp_body(x_ref, w_ref, b_ref, o_ref):
    x = jnp.maximum(x_ref[...], 0.0)
    acc = jax.lax.dot_general(
        x, w_ref[...], (((1,), (0,)), ((), ())),
        preferred_element_type=jnp.float32)
    o_ref[...] = jnp.maximum(acc + b_ref[...], 0.0)


def _mlp(x, w, b):
    B, K = x.shape
    N = w.shape[1]
    BLK = 2048
    return pl.pallas_call(
        _mlp_body,
        grid=(B // BLK,),
        in_specs=[
            pl.BlockSpec((BLK, K), lambda i: (i, 0)),
            pl.BlockSpec((K, N), lambda i: (0, 0)),
            pl.BlockSpec((N,), lambda i: (0,)),
        ],
        out_specs=pl.BlockSpec((BLK, N), lambda i: (i, 0)),
        out_shape=jax.ShapeDtypeStruct((B, N), jnp.float32),
    )(x, w, b)


def _tail_pad(x, cw):
    nfull = x.shape[0] // cw
    d = x.shape[1]
    tail = x[nfull * cw:, :]
    return jnp.pad(tail, ((0, 0), (0, 128 - d)))


def kernel(arg0_1, arg1_1, arg2_1, arg3_1, arg4_1, arg5_1, arg6_1, arg7_1, arg8_1):
    B = arg1_1.shape[0]
    d0, d1, d2 = arg0_1.shape[1], arg2_1.shape[1], arg4_1.shape[1]
    g = _make_gather3(B, d0, arg0_1.shape[0], d1, arg2_1.shape[0],
                      d2, arg4_1.shape[0])
    f0, f1, f2 = g(arg0_1.T, arg1_1, arg2_1.T, arg3_1, arg4_1.T, arg5_1,
                   _tail_pad(arg0_1, 384), _tail_pad(arg2_1, 384),
                   _tail_pad(arg4_1, 128))
    relu_1 = _mlp(arg7_1, arg6_1, arg8_1)
    return (f0[:B, :d0], f1[:B, :d1], f2[:B, :d2], arg6_1, relu_1)
